# baseline (speedup 1.0000x reference)
import functools

import jax
import jax.numpy as jnp
from jax import lax
from jax.experimental import pallas as pl
from jax.experimental.pallas import tpu as pltpu
from jax.experimental.pallas import tpu_sc as plsc

_B = 1024
_V = 100000
_K = 20
_N = _B * _V

_NC = 2
_NS = 16
_NW = _NC * _NS
_L = 16

_PER_W = (_B * _K) // _NW
_NCH = _PER_W // _L
_IDX_ROWS = _PER_W // 128

_GW = 128
_GROWS = _N // _GW

_BR = 16
_GRID = _B // _BR

_PC = (0.6931471596930971, 0.5000011560316415, 0.12498464848034356,
       8.310228184892147e-05, -0.005426855422417802,
       0.00028751330110348837, 0.00018498514140021503)


def _sc_gather_body(src128, tgtf, roff, lrid, out,
                    tgt_v, roff_v, lrid_v, idx_v, lane_v, rows_v, val_v, sem):
    c = lax.axis_index("c")
    s = lax.axis_index("s")
    wid = s * _NC + c
    base = wid * _PER_W
    pltpu.sync_copy(tgtf.at[pl.ds(base, _PER_W)], tgt_v)
    pltpu.sync_copy(roff.at[pl.ds(base, _PER_W)], roff_v)
    pltpu.sync_copy(lrid, lrid_v)
    for ch in range(_NCH):
        sl = pl.ds(ch * _L, _L)
        flat = tgt_v[sl] + roff_v[sl]
        r128 = lax.shift_right_logical(flat, 7)
        lane = lax.bitwise_and(flat, _GW - 1)
        idx_v[ch // 8, pl.ds((ch % 8) * _L, _L)] = r128
        lane_v[sl] = lane
    copies = [
        pltpu.async_copy(src128.at[idx_v.at[q]],
                         rows_v.at[pl.ds(q * 128, 128)], sem)
        for q in range(_IDX_ROWS)
    ]
    for cp in copies:
        cp.wait()
    for ch in range(_NCH):
        sl = pl.ds(ch * _L, _L)
        val_v[sl] = plsc.load_gather(rows_v, [lrid_v[sl], lane_v[sl]])
    pltpu.sync_copy(val_v, out.at[pl.ds(base, _PER_W)])


@functools.cache
def _sc_gather():
    return pl.kernel(
        _sc_gather_body,
        out_type=jax.ShapeDtypeStruct((_B * _K,), jnp.float32),
        compiler_params=pltpu.CompilerParams(needs_layout_passes=False),
        mesh=plsc.VectorSubcoreMesh(
            core_axis_name="c", subcore_axis_name="s",
            num_cores=_NC, num_subcores=_NS),
        scratch_types=[
            pltpu.VMEM((_PER_W,), jnp.int32),
            pltpu.VMEM((_PER_W,), jnp.int32),
            pltpu.VMEM((_PER_W,), jnp.int32),
            pltpu.VMEM((_IDX_ROWS, 128), jnp.int32),
            pltpu.VMEM((_PER_W,), jnp.int32),
            pltpu.VMEM((_PER_W, _GW), jnp.float32),
            pltpu.VMEM((_PER_W,), jnp.float32),
            pltpu.SemaphoreType.DMA,
        ],
    )


def _dense_body(src_ref, tgt_ref, vals_ref, out_ref):
    step = pl.program_id(0)
    x = jnp.clip(src_ref[...], 1e-8, 1.0 - 1e-8)
    g = jnp.float32(_PC[6])
    for c in range(5, -1, -1):
        g = g * x + jnp.float32(_PC[c])

    @pl.when(step == 0)
    def _init():
        out_ref[...] = jnp.zeros_like(out_ref)

    out_ref[...] += jnp.sum(g)

    @pl.when(step == _GRID - 1)
    def _fin():
        t = tgt_ref[...]
        v = jnp.clip(vals_ref[...], 1e-8, 1.0 - 1e-8)
        cols = [t[:, k] for k in range(_K)]
        vcols = [v[:, k] for k in range(_K)]
        corr = jnp.sum(vcols[0])
        for k in range(1, _K):
            keep = cols[0] != cols[k]
            for j in range(1, k):
                keep &= cols[j] != cols[k]
            corr += jnp.sum(jnp.where(keep, vcols[k], 0.0))
        out_ref[...] = (out_ref[...] - corr) * (1.0 / _N)


def _dense_call(src, tgt, vals):
    return pl.pallas_call(
        _dense_body,
        grid=(_GRID,),
        in_specs=[
            pl.BlockSpec((_BR, _V), lambda i: (i, 0)),
            pl.BlockSpec((_B, _K), lambda i: (0, 0)),
            pl.BlockSpec((_B, _K), lambda i: (0, 0)),
        ],
        out_specs=pl.BlockSpec((1, 1), lambda i: (0, 0)),
        out_shape=jax.ShapeDtypeStruct((1, 1), jnp.float32),
    )(src, tgt, vals)


def kernel(src, tgt):
    src128 = src.reshape(_GROWS, _GW)
    tgtf = tgt.reshape(-1).astype(jnp.int32)
    roff = ((jnp.arange(_B * _K, dtype=jnp.int32) // _K) * _V).astype(jnp.int32)
    lrid = jnp.arange(_PER_W, dtype=jnp.int32)
    vals = _sc_gather()(src128, tgtf, roff, lrid).reshape(_B, _K)
    out = _dense_call(src, tgt.astype(jnp.int32), vals)
    return out[0, 0]


# speedup vs baseline: 2.4558x; 2.4558x over previous
import functools

import jax
import jax.numpy as jnp
from jax import lax
from jax.experimental import pallas as pl
from jax.experimental.pallas import tpu as pltpu
from jax.experimental.pallas import tpu_sc as plsc

_B = 1024
_V = 100000
_K = 20
_N = _B * _V

_NC = 2
_NS = 16
_NW = _NC * _NS
_L = 16

_PER_W = (_B * _K) // _NW
_NCH = _PER_W // _L
_BATCH = 64
_NBATCH = _PER_W // _BATCH

_BR = 1000
_GRID = _V // _BR

_PC = (0.6931471596930971, 0.5000011560316415, 0.12498464848034356,
       8.310228184892147e-05, -0.005426855422417802,
       0.00028751330110348837, 0.00018498514140021503)


def _sc_gather_body(srcT, tgtf, rid, lrid, out,
                    tgt_v, rid_v, lrid_v, idx_v, rows_v, val_v, sem):
    c = lax.axis_index("c")
    s = lax.axis_index("s")
    wid = s * _NC + c
    base = wid * _PER_W
    pltpu.sync_copy(tgtf.at[pl.ds(base, _PER_W)], tgt_v)
    pltpu.sync_copy(rid.at[pl.ds(base, _PER_W)], rid_v)
    pltpu.sync_copy(lrid, lrid_v)
    for ch in range(_NCH):
        t = tgt_v[pl.ds(ch * _L, _L)]
        idx_v[ch // 4, pl.ds((ch % 4) * _L, _L)] = t
    for b in range(_NBATCH):
        pltpu.async_copy(srcT.at[idx_v.at[b]], rows_v, sem).wait()
        for q in range(_BATCH // _L):
            sl = pl.ds((b * (_BATCH // _L) + q) * _L, _L)
            val_v[sl] = plsc.load_gather(rows_v, [lrid_v[sl], rid_v[sl]])
    pltpu.sync_copy(val_v, out.at[pl.ds(base, _PER_W)])


@functools.cache
def _sc_gather():
    return pl.kernel(
        _sc_gather_body,
        out_type=jax.ShapeDtypeStruct((_B * _K,), jnp.float32),
        compiler_params=pltpu.CompilerParams(needs_layout_passes=False),
        mesh=plsc.VectorSubcoreMesh(
            core_axis_name="c", subcore_axis_name="s",
            num_cores=_NC, num_subcores=_NS),
        scratch_types=[
            pltpu.VMEM((_PER_W,), jnp.int32),
            pltpu.VMEM((_PER_W,), jnp.int32),
            pltpu.VMEM((_PER_W,), jnp.int32),
            pltpu.VMEM((_NBATCH, _BATCH), jnp.int32),
            pltpu.VMEM((_BATCH, _B), jnp.float32),
            pltpu.VMEM((_PER_W,), jnp.float32),
            pltpu.SemaphoreType.DMA,
        ],
    )


def _dense_body(src_ref, out_ref):
    step = pl.program_id(0)
    x = jnp.clip(src_ref[...], 1e-8, 1.0 - 1e-8)
    g = jnp.float32(_PC[6])
    for c in range(5, -1, -1):
        g = g * x + jnp.float32(_PC[c])

    @pl.when(step == 0)
    def _init():
        out_ref[...] = jnp.zeros_like(out_ref)

    out_ref[...] += jnp.sum(g)


def _dense_call(srcT):
    return pl.pallas_call(
        _dense_body,
        grid=(_GRID,),
        in_specs=[pl.BlockSpec((_BR, _B), lambda i: (i, 0))],
        out_specs=pl.BlockSpec((1, 1), lambda i: (0, 0)),
        out_shape=jax.ShapeDtypeStruct((1, 1), jnp.float32),
    )(srcT)


def _combine_body(tgt_ref, vals_ref, dsum_ref, out_ref):
    t = tgt_ref[...]
    v = jnp.clip(vals_ref[...], 1e-8, 1.0 - 1e-8)
    cols = [t[:, k] for k in range(_K)]
    vcols = [v[:, k] for k in range(_K)]
    corr = jnp.sum(vcols[0])
    for k in range(1, _K):
        keep = cols[0] != cols[k]
        for j in range(1, k):
            keep &= cols[j] != cols[k]
        corr += jnp.sum(jnp.where(keep, vcols[k], 0.0))
    out_ref[...] = (dsum_ref[...] - corr) * (1.0 / _N)


def _combine_call(tgt, vals, dsum):
    return pl.pallas_call(
        _combine_body,
        out_shape=jax.ShapeDtypeStruct((1, 1), jnp.float32),
    )(tgt, vals, dsum)


def kernel(src, tgt):
    srcT = src.T
    tgtf = tgt.reshape(-1).astype(jnp.int32)
    rid = (jnp.arange(_B * _K, dtype=jnp.int32) // _K).astype(jnp.int32)
    lrid = (jnp.arange(_PER_W, dtype=jnp.int32) % _BATCH).astype(jnp.int32)
    vals = _sc_gather()(srcT, tgtf, rid, lrid).reshape(_B, _K)
    dsum = _dense_call(srcT)
    out = _combine_call(tgt.astype(jnp.int32), vals, dsum)
    return out[0, 0]


# speedup vs baseline: 4.1473x; 1.6888x over previous
import functools

import jax
import jax.numpy as jnp
from jax import lax
from jax.experimental import pallas as pl
from jax.experimental.pallas import tpu as pltpu
from jax.experimental.pallas import tpu_sc as plsc

_B = 1024
_V = 100000
_K = 20
_N = _B * _V

_NC = 2
_NS = 16
_NW = _NC * _NS
_L = 16

_PER_W = (_B * _K) // _NW
_NCH = _PER_W // _L
_BATCH = 32
_NBATCH = _PER_W // _BATCH

_SCH_ROWS = 32
_SCH = 44
_SROWS_PT = _SCH_ROWS * _SCH
_SPLIT = _V - _SROWS_PT * _NW

_BR = 808
_GRID = _SPLIT // _BR

_PC = (0.6931471596930971, 0.5000011560316415, 0.12498464848034356,
       8.310228184892147e-05, -0.005426855422417802,
       0.00028751330110348837, 0.00018498514140021503)


def _poly16(x):
    x = jnp.clip(x, jnp.float32(1e-8), jnp.float32(1.0 - 1e-8))
    g = jnp.float32(_PC[6])
    for c in range(5, -1, -1):
        g = g * x + jnp.float32(_PC[c])
    return g


def _sc_gather_body(srcT, tgtf, rid, lrid, out, psums,
                    tgt_v, rid_v, lrid_v, idx_v, rows_v, buf2, val_v, acc_v,
                    sem, sem2):
    c = lax.axis_index("c")
    s = lax.axis_index("s")
    wid = s * _NC + c
    base = wid * _PER_W
    pltpu.sync_copy(tgtf.at[pl.ds(base, _PER_W)], tgt_v)
    pltpu.sync_copy(rid.at[pl.ds(base, _PER_W)], rid_v)
    pltpu.sync_copy(lrid, lrid_v)
    for ch in range(_NCH):
        t = tgt_v[pl.ds(ch * _L, _L)]
        idx_v[ch // 2, pl.ds((ch % 2) * _L, _L)] = t
    for b in range(_NBATCH):
        pltpu.async_copy(srcT.at[idx_v.at[b]], rows_v, sem).wait()
        for q in range(_BATCH // _L):
            sl = pl.ds((b * (_BATCH // _L) + q) * _L, _L)
            val_v[sl] = plsc.load_gather(rows_v, [lrid_v[sl], rid_v[sl]])
    pltpu.sync_copy(val_v, out.at[pl.ds(base, _PER_W)])

    rbase = _SPLIT + wid * _SROWS_PT

    def _rows0(r, a):
        for q in range(_B // _L):
            a += _poly16(rows_v[r, pl.ds(q * _L, _L)])
        return a

    def _rows1(r, a):
        for q in range(_B // _L):
            a += _poly16(buf2[r, pl.ds(q * _L, _L)])
        return a

    def _pair_body(i, acc):
        off1 = rbase + (2 * i + 1) * _SCH_ROWS
        pltpu.async_copy(srcT.at[pl.ds(off1, _SCH_ROWS)], buf2, sem2)
        pltpu.make_async_copy(
            srcT.at[pl.ds(rbase, _SCH_ROWS)], rows_v, sem).wait()
        acc = lax.fori_loop(0, _SCH_ROWS, _rows0, acc)
        off2 = rbase + jnp.minimum(2 * i + 2, _SCH - 1) * _SCH_ROWS
        pltpu.async_copy(srcT.at[pl.ds(off2, _SCH_ROWS)], rows_v, sem)
        pltpu.make_async_copy(
            srcT.at[pl.ds(rbase, _SCH_ROWS)], buf2, sem2).wait()
        return lax.fori_loop(0, _SCH_ROWS, _rows1, acc)

    pltpu.async_copy(srcT.at[pl.ds(rbase, _SCH_ROWS)], rows_v, sem)
    acc = lax.fori_loop(0, _SCH // 2, _pair_body,
                        jnp.zeros((_L,), jnp.float32))
    pltpu.make_async_copy(
        srcT.at[pl.ds(rbase, _SCH_ROWS)], rows_v, sem).wait()
    acc_v[pl.ds(0, _L)] = acc
    pltpu.sync_copy(acc_v, psums.at[wid])


@functools.cache
def _sc_gather():
    return pl.kernel(
        _sc_gather_body,
        out_type=[jax.ShapeDtypeStruct((_B * _K,), jnp.float32),
                  jax.ShapeDtypeStruct((_NW, _L), jnp.float32)],
        compiler_params=pltpu.CompilerParams(needs_layout_passes=False),
        mesh=plsc.VectorSubcoreMesh(
            core_axis_name="c", subcore_axis_name="s",
            num_cores=_NC, num_subcores=_NS),
        scratch_types=[
            pltpu.VMEM((_PER_W,), jnp.int32),
            pltpu.VMEM((_PER_W,), jnp.int32),
            pltpu.VMEM((_PER_W,), jnp.int32),
            pltpu.VMEM((_NBATCH, _BATCH), jnp.int32),
            pltpu.VMEM((_SCH_ROWS, _B), jnp.float32),
            pltpu.VMEM((_SCH_ROWS, _B), jnp.float32),
            pltpu.VMEM((_PER_W,), jnp.float32),
            pltpu.VMEM((_L,), jnp.float32),
            pltpu.SemaphoreType.DMA,
            pltpu.SemaphoreType.DMA,
        ],
    )


def _dense_body(src_ref, out_ref):
    step = pl.program_id(0)
    x = jnp.clip(src_ref[...], 1e-8, 1.0 - 1e-8)
    g = jnp.float32(_PC[6])
    for c in range(5, -1, -1):
        g = g * x + jnp.float32(_PC[c])

    @pl.when(step == 0)
    def _init():
        out_ref[...] = jnp.zeros_like(out_ref)

    out_ref[...] += jnp.sum(g)


def _dense_call(srcT):
    return pl.pallas_call(
        _dense_body,
        grid=(_GRID,),
        in_specs=[pl.BlockSpec((_BR, _B), lambda i: (i, 0))],
        out_specs=pl.BlockSpec((1, 1), lambda i: (0, 0)),
        out_shape=jax.ShapeDtypeStruct((1, 1), jnp.float32),
    )(srcT)


def _combine_body(tgt_ref, vals_ref, dsum_ref, psums_ref, out_ref):
    t = tgt_ref[...]
    v = jnp.clip(vals_ref[...], 1e-8, 1.0 - 1e-8)
    cols = [t[:, k] for k in range(_K)]
    vcols = [v[:, k] for k in range(_K)]
    corr = jnp.sum(vcols[0])
    for k in range(1, _K):
        keep = cols[0] != cols[k]
        for j in range(1, k):
            keep &= cols[j] != cols[k]
        corr += jnp.sum(jnp.where(keep, vcols[k], 0.0))
    total = dsum_ref[...] + jnp.sum(psums_ref[...]) - corr
    out_ref[...] = total * (1.0 / _N)


def _combine_call(tgt, vals, dsum, psums):
    return pl.pallas_call(
        _combine_body,
        out_shape=jax.ShapeDtypeStruct((1, 1), jnp.float32),
    )(tgt, vals, dsum, psums)


def kernel(src, tgt):
    srcT = src.T
    tgtf = tgt.reshape(-1).astype(jnp.int32)
    rid = (jnp.arange(_B * _K, dtype=jnp.int32) // _K).astype(jnp.int32)
    lrid = (jnp.arange(_PER_W, dtype=jnp.int32) % _BATCH).astype(jnp.int32)
    vals_flat, psums = _sc_gather()(srcT, tgtf, rid, lrid)
    vals = vals_flat.reshape(_B, _K)
    dsum = _dense_call(srcT)
    out = _combine_call(tgt.astype(jnp.int32), vals, dsum, psums)
    return out[0, 0]
